# baseline (device time: 576145 ns/iter reference)
import jax
import jax.numpy as jnp
from jax import lax
from jax.experimental import pallas as pl
from jax.experimental.pallas import tpu as pltpu

N_DEV = 32


def kernel(x, k, Wp):
    b, s, c = x.shape
    taps = k.shape[0]
    n = Wp.shape[1]

    def body(x_ref, k_ref, w_ref, out_ref, comm_ref, send_sems, recv_sems,
             credit_sem):
        my = lax.axis_index("i")
        left = lax.rem(my - 1 + N_DEV, N_DEV)
        right = lax.rem(my + 1, N_DEV)

        barrier_sem = pltpu.get_barrier_semaphore()
        for nbr in (left, right):
            pl.semaphore_signal(
                barrier_sem, inc=1,
                device_id=(nbr,), device_id_type=pl.DeviceIdType.MESH,
            )
        pl.semaphore_wait(barrier_sem, 2)

        xv = x_ref[...]
        kv = k_ref[...]
        o = xv * kv[taps - 1].reshape(1, 1, c)
        for t in range(taps - 1):
            shift = taps - 1 - t
            shifted = jnp.concatenate(
                [jnp.zeros((b, shift, c), xv.dtype), xv[:, : s - shift, :]],
                axis=1,
            )
            o = o + shifted * kv[t].reshape(1, 1, c)
        a = (o * jax.nn.sigmoid(o)).astype(jnp.bfloat16)
        w = w_ref[...].astype(jnp.bfloat16)
        partial = lax.dot_general(
            a.reshape(b * s, c), w,
            dimension_numbers=(((1,), (0,)), ((), ())),
            preferred_element_type=jnp.float32,
        )

        comm_ref[0] = partial.astype(jnp.bfloat16)
        acc = partial

        for h in range(N_DEV - 1):
            send_slot = h % 2
            recv_slot = (h + 1) % 2
            if h >= 2:
                pl.semaphore_wait(credit_sem, 1)
            rdma = pltpu.make_async_remote_copy(
                src_ref=comm_ref.at[send_slot],
                dst_ref=comm_ref.at[recv_slot],
                send_sem=send_sems.at[send_slot],
                recv_sem=recv_sems.at[recv_slot],
                device_id=(right,),
                device_id_type=pl.DeviceIdType.MESH,
            )
            rdma.start()
            rdma.wait()
            acc = acc + comm_ref[recv_slot].astype(jnp.float32)
            if 1 <= h <= N_DEV - 3:
                pl.semaphore_signal(
                    credit_sem, inc=1,
                    device_id=(left,), device_id_type=pl.DeviceIdType.MESH,
                )

        out_ref[...] = acc.reshape(b, s, n)

    return pl.pallas_call(
        body,
        out_shape=jax.ShapeDtypeStruct((b, s, n), jnp.float32),
        in_specs=[
            pl.BlockSpec(memory_space=pltpu.VMEM),
            pl.BlockSpec(memory_space=pltpu.VMEM),
            pl.BlockSpec(memory_space=pltpu.VMEM),
        ],
        out_specs=pl.BlockSpec(memory_space=pltpu.VMEM),
        scratch_shapes=[
            pltpu.VMEM((2, b * s, n), jnp.bfloat16),
            pltpu.SemaphoreType.DMA((2,)),
            pltpu.SemaphoreType.DMA((2,)),
            pltpu.SemaphoreType.REGULAR,
        ],
        compiler_params=pltpu.CompilerParams(collective_id=0),
    )(x, k, Wp)


# device time: 41655 ns/iter; 13.8314x vs baseline; 13.8314x over previous
import jax
import jax.numpy as jnp
from jax import lax
from jax.experimental import pallas as pl
from jax.experimental.pallas import tpu as pltpu

N_DEV = 32


def kernel(x, k, Wp):
    b, s, c = x.shape
    taps = k.shape[0]
    n = Wp.shape[1]
    rows = b * s
    chunk = rows // N_DEV

    def body(x_ref, k_ref, w_ref, out_ref, src_buf, rs_buf, ag_buf,
             rs_send, rs_recv, ag_send, ag_recv):
        my = lax.axis_index("i")

        barrier_sem = pltpu.get_barrier_semaphore()
        for r in range(1, N_DEV):
            tgt = lax.rem(my + r, N_DEV)
            pl.semaphore_signal(
                barrier_sem, inc=1,
                device_id=(tgt,), device_id_type=pl.DeviceIdType.MESH,
            )
        pl.semaphore_wait(barrier_sem, N_DEV - 1)

        xv = x_ref[...]
        kv = k_ref[...]
        o = xv * kv[taps - 1].reshape(1, 1, c)
        for t in range(taps - 1):
            shift = taps - 1 - t
            shifted = jnp.concatenate(
                [jnp.zeros((b, shift, c), xv.dtype), xv[:, : s - shift, :]],
                axis=1,
            )
            o = o + shifted * kv[t].reshape(1, 1, c)
        a = (o * jax.nn.sigmoid(o)).astype(jnp.bfloat16)
        w = w_ref[...].astype(jnp.bfloat16)
        partial = lax.dot_general(
            a.reshape(rows, c), w,
            dimension_numbers=(((1,), (0,)), ((), ())),
            preferred_element_type=jnp.float32,
        )

        src_buf[...] = partial.astype(jnp.bfloat16).reshape(N_DEV, chunk, n)

        rs_buf[0] = src_buf[my]
        rs_rdmas = []
        for r in range(1, N_DEV):
            tgt = lax.rem(my + r, N_DEV)
            rdma = pltpu.make_async_remote_copy(
                src_ref=src_buf.at[tgt],
                dst_ref=rs_buf.at[N_DEV - r],
                send_sem=rs_send.at[r],
                recv_sem=rs_recv.at[N_DEV - r],
                device_id=(tgt,),
                device_id_type=pl.DeviceIdType.MESH,
            )
            rdma.start()
            rs_rdmas.append(rdma)

        acc = rs_buf[0].astype(jnp.float32)
        for r in range(1, N_DEV):
            rs_rdmas[r - 1].wait_recv()
            acc = acc + rs_buf[N_DEV - r].astype(jnp.float32)

        ag_buf[my] = acc.astype(jnp.bfloat16)
        ag_rdmas = []
        for r in range(1, N_DEV):
            tgt = lax.rem(my + r, N_DEV)
            rdma = pltpu.make_async_remote_copy(
                src_ref=ag_buf.at[my],
                dst_ref=ag_buf.at[my],
                send_sem=ag_send.at[r],
                recv_sem=ag_recv.at[r],
                device_id=(tgt,),
                device_id_type=pl.DeviceIdType.MESH,
            )
            rdma.start()
            ag_rdmas.append(rdma)

        for r in range(1, N_DEV):
            rs_rdmas[r - 1].wait_send()

        for r in range(1, N_DEV):
            ag_rdmas[r - 1].wait_recv()

        out_ref[...] = ag_buf[...].astype(jnp.float32).reshape(b, s, n)

        for r in range(1, N_DEV):
            ag_rdmas[r - 1].wait_send()

    return pl.pallas_call(
        body,
        out_shape=jax.ShapeDtypeStruct((b, s, n), jnp.float32),
        in_specs=[
            pl.BlockSpec(memory_space=pltpu.VMEM),
            pl.BlockSpec(memory_space=pltpu.VMEM),
            pl.BlockSpec(memory_space=pltpu.VMEM),
        ],
        out_specs=pl.BlockSpec(memory_space=pltpu.VMEM),
        scratch_shapes=[
            pltpu.VMEM((N_DEV, chunk, n), jnp.bfloat16),
            pltpu.VMEM((N_DEV, chunk, n), jnp.bfloat16),
            pltpu.VMEM((N_DEV, chunk, n), jnp.bfloat16),
            pltpu.SemaphoreType.DMA((N_DEV,)),
            pltpu.SemaphoreType.DMA((N_DEV,)),
            pltpu.SemaphoreType.DMA((N_DEV,)),
            pltpu.SemaphoreType.DMA((N_DEV,)),
        ],
        compiler_params=pltpu.CompilerParams(collective_id=0),
    )(x, k, Wp)
